# baseline (speedup 1.0000x reference)
import jax
import jax.numpy as jnp
from jax.experimental import pallas as pl
from jax.experimental.pallas import tpu as pltpu

_N = 100000
_S = 64
_B = 1000
_NB = _N // _B


def _body(idx_ref, out_ref):
    ids = idx_ref[0, 0, :]
    iot = jax.lax.broadcasted_iota(jnp.int32, (_B, _S), 1)
    out_ref[...] = (ids[:, None] == iot).astype(jnp.float32)


@jax.jit
def _onehot_tc(species_index):
    idx3 = species_index.reshape(_NB, 1, _B)
    return pl.pallas_call(
        _body,
        grid=(_NB,),
        in_specs=[pl.BlockSpec((1, 1, _B), lambda i: (i, 0, 0))],
        out_specs=pl.BlockSpec((_B, _S), lambda i: (i, 0)),
        out_shape=jax.ShapeDtypeStruct((_N, _S), jnp.float32),
    )(idx3)


def kernel(species_index, pos):
    return _onehot_tc(species_index)


# speedup vs baseline: 3.1123x; 3.1123x over previous
import jax
import jax.numpy as jnp
from jax import lax
from jax.experimental import pallas as pl
from jax.experimental.pallas import tpu as pltpu
from jax.experimental.pallas import tpu_sc as plsc

_N = 100000
_S = 64
_C = 640
_KF = _N // _C
_CT = _N - _KF * _C
_NW = 32
_T = (_KF + _NW - 1) // _NW
_G = _C // 16
_GT = _CT // 16


def _body(idx_hbm, out_hbm, idx_v, buf_v, tidx_v, tbuf_v, isems, osems, tsem):
    c = lax.axis_index("c")
    s = lax.axis_index("s")
    w = s * 2 + c

    lanes = lax.iota(jnp.int32, 16)
    ones = jnp.full((16,), 1.0, jnp.float32)
    zeros = jnp.zeros((16,), jnp.float32)

    cids = [jnp.minimum(w + _NW * t, _KF - 1) for t in range(_T)]

    idx_dmas = []
    for t in range(_T):
        dma = pltpu.make_async_copy(
            idx_hbm.at[pl.ds(cids[t] * _C, _C)], idx_v[t], isems[t]
        )
        dma.start()
        idx_dmas.append(dma)

    def _zero(b, ng):
        def zloop(r, carry):
            rows = lanes * 0 + r
            for u in range(ng):
                plsc.store_scatter(b, [rows, lanes + 16 * u], zeros)
            return carry

        lax.fori_loop(0, _S, zloop, None)

    _zero(buf_v[0], _G)
    _zero(buf_v[1], _G)

    def _scatter(buf, idx, val, ng):
        def loop(i, carry):
            ids = idx[pl.ds(i * 16, 16)]
            plsc.store_scatter(buf, [ids, i * 16 + lanes], val)
            return carry

        lax.fori_loop(0, ng, loop, None)

    out_dmas = [None, None]
    for t in range(_T):
        b = t % 2
        if t >= 2:
            out_dmas[b].wait()
            _scatter(buf_v[b], idx_v[t - 2], zeros, _G)
        idx_dmas[t].wait()
        _scatter(buf_v[b], idx_v[t], ones, _G)
        out_dmas[b] = pltpu.make_async_copy(
            buf_v[b], out_hbm.at[:, pl.ds(cids[t] * _C, _C)], osems[b]
        )
        out_dmas[b].start()

    @pl.when(w == _NW - 1)
    def _tail():
        pltpu.sync_copy(idx_hbm.at[pl.ds(_KF * _C, _CT)], tidx_v)
        _zero(tbuf_v, _GT)
        _scatter(tbuf_v, tidx_v, ones, _GT)
        tdma = pltpu.make_async_copy(
            tbuf_v, out_hbm.at[:, pl.ds(_KF * _C, _CT)], tsem
        )
        tdma.start()
        tdma.wait()

    out_dmas[0].wait()
    out_dmas[1].wait()


@jax.jit
def _onehot_sc(species_index):
    mesh = plsc.VectorSubcoreMesh(core_axis_name="c", subcore_axis_name="s")
    f = pl.kernel(
        _body,
        out_type=jax.ShapeDtypeStruct((_S, _N), jnp.float32),
        mesh=mesh,
        compiler_params=pltpu.CompilerParams(needs_layout_passes=False),
        scratch_types=[
            [pltpu.VMEM((_C,), jnp.int32) for _ in range(_T)],
            [pltpu.VMEM((_S, _C), jnp.float32) for _ in range(2)],
            pltpu.VMEM((_CT,), jnp.int32),
            pltpu.VMEM((_S, _CT), jnp.float32),
            [pltpu.SemaphoreType.DMA for _ in range(_T)],
            [pltpu.SemaphoreType.DMA for _ in range(2)],
            pltpu.SemaphoreType.DMA,
        ],
    )
    return f(species_index)


def kernel(species_index, pos):
    return _onehot_sc(species_index).T


# speedup vs baseline: 3.2267x; 1.0367x over previous
import jax
import jax.numpy as jnp
from jax import lax
from jax.experimental import pallas as pl
from jax.experimental.pallas import tpu as pltpu
from jax.experimental.pallas import tpu_sc as plsc

_N = 100000
_S = 64
_C = 640
_KF = _N // _C
_CT = _N - _KF * _C
_NW = 32
_T = (_KF + _NW - 1) // _NW
_G = _C // 16
_GT = _CT // 16


def _body(idx_hbm, out_hbm, idx_v, buf_v, tidx_v, tbuf_v, isems, osems, tsem):
    c = lax.axis_index("c")
    s = lax.axis_index("s")
    w = s * 2 + c

    lanes = lax.iota(jnp.int32, 16)
    ones = jnp.full((16,), 1.0, jnp.float32)
    zeros = jnp.zeros((16,), jnp.float32)

    cids = [jnp.minimum(w + _NW * t, _KF - 1) for t in range(_T)]

    idx_dmas = []
    for t in range(_T):
        dma = pltpu.make_async_copy(
            idx_hbm.at[pl.ds(cids[t] * _C, _C)], idx_v[t], isems[t]
        )
        dma.start()
        idx_dmas.append(dma)

    def _zero(b, ng):
        def zloop(r, carry):
            rows = lanes * 0 + r
            for u in range(ng):
                plsc.store_scatter(b, [rows, lanes + 16 * u], zeros)
            return carry

        lax.fori_loop(0, _S, zloop, None)

    _zero(buf_v[0], _G)

    def _scatter(buf, idx, val, ng):
        def loop(i, carry):
            ids = idx[pl.ds(i * 16, 16)]
            plsc.store_scatter(buf, [ids, i * 16 + lanes], val)
            return carry

        lax.fori_loop(0, ng, loop, None)

    out_dmas = [None, None]
    for t in range(_T):
        b = t % 2
        if t == 1:
            _zero(buf_v[1], _G)
        if t >= 2:
            out_dmas[b].wait()
            _scatter(buf_v[b], idx_v[t - 2], zeros, _G)
        idx_dmas[t].wait()
        _scatter(buf_v[b], idx_v[t], ones, _G)
        out_dmas[b] = pltpu.make_async_copy(
            buf_v[b], out_hbm.at[:, pl.ds(cids[t] * _C, _C)], osems[b]
        )
        out_dmas[b].start()

    @pl.when(w == _NW - 1)
    def _tail():
        pltpu.sync_copy(idx_hbm.at[pl.ds(_KF * _C, _CT)], tidx_v)
        _zero(tbuf_v, _GT)
        _scatter(tbuf_v, tidx_v, ones, _GT)
        tdma = pltpu.make_async_copy(
            tbuf_v, out_hbm.at[:, pl.ds(_KF * _C, _CT)], tsem
        )
        tdma.start()
        tdma.wait()

    out_dmas[0].wait()
    out_dmas[1].wait()


@jax.jit
def _onehot_sc(species_index):
    mesh = plsc.VectorSubcoreMesh(core_axis_name="c", subcore_axis_name="s")
    f = pl.kernel(
        _body,
        out_type=jax.ShapeDtypeStruct((_S, _N), jnp.float32),
        mesh=mesh,
        compiler_params=pltpu.CompilerParams(needs_layout_passes=False),
        scratch_types=[
            [pltpu.VMEM((_C,), jnp.int32) for _ in range(_T)],
            [pltpu.VMEM((_S, _C), jnp.float32) for _ in range(2)],
            pltpu.VMEM((_CT,), jnp.int32),
            pltpu.VMEM((_S, _CT), jnp.float32),
            [pltpu.SemaphoreType.DMA for _ in range(_T)],
            [pltpu.SemaphoreType.DMA for _ in range(2)],
            pltpu.SemaphoreType.DMA,
        ],
    )
    return f(species_index)


def kernel(species_index, pos):
    return _onehot_sc(species_index).T
